# Initial kernel scaffold; baseline (speedup 1.0000x reference)
import jax
import jax.numpy as jnp
from jax.experimental import pallas as pl


def kernel(pd_scores, pd_bboxes, anc_points, gt_labels, gt_bboxes, gt_mask):
    raise NotImplementedError("write your pallas kernel here")


# speedup vs baseline: 18.5072x; 18.5072x over previous
import math

import jax
import jax.numpy as jnp
from jax.experimental import pallas as pl
from jax.experimental.pallas import tpu as pltpu

TOPK = 13
NC = 80
EPS_IN = 1e-9
EPS_IOU = 1e-7

_INTERPRET = False

_ATAN_COEFFS = (
    9.999999999776e-01, -3.333333207911e-01, 1.999992819090e-01,
    -1.428419858875e-01, 1.109481209099e-01, -8.987009293820e-02,
    7.264201452628e-02, -5.460683193113e-02, 3.458600582713e-02,
    -1.638182583930e-02, 4.961051519167e-03, -7.042539361997e-04,
)


def _atan_pos(x):
    inv = x > 1.0
    z = jnp.where(inv, 1.0 / x, x)
    u = z * z
    p = jnp.full_like(u, _ATAN_COEFFS[-1])
    for c in _ATAN_COEFFS[-2::-1]:
        p = p * u + c
    at = z * p
    return jnp.where(inv, (math.pi / 2) - at, at)


def _body(ps_ref, pb_ref, anc_ref, gl_ref, gtb_ref, gm_ref,
          cls_ref, tb_ref, ts_ref, fm_ref, tg_ref,
          work_ref, sel_ref, isin_ref, ncm_ref, cfl_ref):
    p = pl.program_id(0)
    b = pl.program_id(1)
    bs = pl.num_programs(1)
    ngt = gtb_ref.shape[1]
    na = pb_ref.shape[2]

    gl = gl_ref[0]
    gtb = gtb_ref[0]
    valid = gm_ref[0] > 0.0
    i_iota = jax.lax.broadcasted_iota(jnp.int32, (ngt, na), 1)
    j_iota = jax.lax.broadcasted_iota(jnp.int32, (ngt, na), 0)

    @pl.when(p == 0)
    def _phase0():
        pb = pb_ref[0]
        px1 = pb[0:1, :]
        px2 = pb[1:2, :]
        py1 = pb[2:3, :]
        py2 = pb[3:4, :]
        anc = anc_ref[0]
        ax = anc[0:1, :]
        ay = anc[1:2, :]
        gx1 = gtb[:, 0:1]
        gx2 = gtb[:, 1:2]
        gy1 = gtb[:, 2:3]
        gy2 = gtb[:, 3:4]

        d1 = ax - gx1
        d2 = ay - gx2
        d3 = gy1 - ax
        d4 = gy2 - ay
        dmin = jnp.minimum(jnp.minimum(d1, d2), jnp.minimum(d3, d4))
        mask = jnp.logical_and(dmin > EPS_IN, valid)

        inter = (jnp.clip(jnp.minimum(px2, gx2) - jnp.maximum(px1, gx1), 0.0, None)
                 * jnp.clip(jnp.minimum(py2, gy2) - jnp.maximum(py1, gy1), 0.0, None))
        w1 = px2 - px1
        h1 = py2 - py1 + EPS_IOU
        w2 = gx2 - gx1
        h2 = gy2 - gy1 + EPS_IOU
        union = w1 * h1 + w2 * h2 - inter + EPS_IOU
        iou = inter / union
        cw = jnp.maximum(px2, gx2) - jnp.minimum(px1, gx1)
        ch = jnp.maximum(py2, gy2) - jnp.minimum(py1, gy1)
        c2 = cw * cw + ch * ch + EPS_IOU
        rho2 = ((gx1 + gx2 - px1 - px2) ** 2 + (gy1 + gy2 - py1 - py2) ** 2) / 4.0
        at1 = _atan_pos(w1 / h1)
        at2 = _atan_pos(w2 / h2)
        dat = at2 - at1
        v = (4.0 / math.pi ** 2) * dat * dat
        alpha = v / (v - iou + (1.0 + EPS_IOU))
        ciou = iou - (rho2 / c2 + v * alpha)
        iou_c = jnp.maximum(ciou, 0.0)

        ps = ps_ref[0]
        cls_iota = jax.lax.broadcasted_iota(jnp.int32, ps.shape, 1)
        oh = (cls_iota == gl).astype(ps.dtype)
        s = jnp.sum(ps * oh, axis=1, keepdims=True)

        i2 = iou_c * iou_c
        i6 = i2 * i2 * i2
        align = s * i6 * mask.astype(ps.dtype)

        work_ref[:] = align
        sel_ref[:] = jnp.zeros(align.shape, jnp.float32)

        def topk_step(_, carry):
            w = work_ref[:]
            m = jnp.max(w, axis=1, keepdims=True)
            eq = w == m
            first = jnp.min(jnp.where(eq, i_iota, na), axis=1, keepdims=True)
            onehot = i_iota == first
            sel_ref[:] = jnp.where(onehot, 1.0, sel_ref[:])
            work_ref[:] = jnp.where(onehot, -jnp.inf, w)
            return carry

        jax.lax.fori_loop(0, TOPK, topk_step, 0)
        sel = sel_ref[:] > 0.0

        is_in = (sel & valid & (align > 1e-9)).astype(jnp.int32)

        cnt = jnp.sum(is_in, axis=0, keepdims=True)
        conflict = cnt > 1
        amax = jnp.max(align, axis=0, keepdims=True)
        cg = jnp.min(jnp.where(align == amax, j_iota, ngt), axis=0,
                     keepdims=True)
        hit = jnp.max((cg == j_iota).astype(jnp.int32), axis=1,
                      keepdims=True)
        ncm_row = jnp.max(jnp.where(i_iota == j_iota, hit, 0), axis=0,
                          keepdims=True)

        isin_ref[pl.ds(b * ngt, ngt), :] = is_in.astype(jnp.int8)
        ncm_ref[pl.ds(b * 8, 8), :] = jnp.broadcast_to(ncm_row, (8, na))
        cfl_ref[b] = jnp.max(conflict.astype(jnp.int32))

    @pl.when(p == 1)
    def _phase1():
        is_in = isin_ref[pl.ds(b * ngt, ngt), :].astype(jnp.int32)
        ncm_row = ncm_ref[pl.ds(b * 8, 1), :]
        cnt = jnp.sum(is_in, axis=0, keepdims=True)
        conflict = cnt > 1
        nm = jnp.logical_not(conflict).astype(jnp.int32)
        ncm = jnp.where(j_iota == 0, ncm_row, 0)
        resolved = (is_in + ncm) * nm
        anycf = jnp.int32(0)
        for bb in range(bs):
            anycf = anycf + cfl_ref[bb]
        is_f = jnp.where(anycf > 0, resolved, is_in)

        fmask = jnp.sum(is_f, axis=0, keepdims=True) > 0
        mxv = jnp.max(is_f, axis=0, keepdims=True)
        tg = jnp.min(jnp.where(is_f == mxv, j_iota, ngt), axis=0,
                     keepdims=True)

        sel_t = j_iota == tg
        cmp_t = sel_t.astype(jnp.float32)
        glf = gl.astype(jnp.float32)
        tcls = jnp.sum(jnp.where(sel_t, glf, 0.0), axis=0, keepdims=True)

        cls_ref[0] = tcls.astype(jnp.int32)
        fm_ref[0] = fmask.astype(jnp.int32)
        tg_ref[0] = tg

        dn = (((0,), (0,)), ((), ()))
        tb_ref[0] = jax.lax.dot_general(cmp_t, gtb, dn,
                                        preferred_element_type=jnp.float32)
        oh_cls = (jax.lax.broadcasted_iota(jnp.int32, (ngt, NC), 1)
                  == gl).astype(jnp.float32)
        ts_ref[0] = jax.lax.dot_general(cmp_t, oh_cls, dn,
                                        preferred_element_type=jnp.float32)


@jax.jit
def kernel(pd_scores, pd_bboxes, anc_points, gt_labels, gt_bboxes, gt_mask):
    bs, na, nc = pd_scores.shape
    ngt = gt_bboxes.shape[1]

    ps_s = pd_scores[:, :ngt, :]
    pb_t = jnp.transpose(pd_bboxes, (0, 2, 1))
    anc_t = jnp.transpose(anc_points).reshape(1, 2, na)
    gl = gt_labels.astype(jnp.int32)

    out_shapes = (
        jax.ShapeDtypeStruct((bs, 1, na), jnp.int32),
        jax.ShapeDtypeStruct((bs, na, 4), jnp.float32),
        jax.ShapeDtypeStruct((bs, na, NC), jnp.float32),
        jax.ShapeDtypeStruct((bs, 1, na), jnp.int32),
        jax.ShapeDtypeStruct((bs, 1, na), jnp.int32),
    )
    grid = (2, bs)

    def in_map(p, b):
        return (b, 0, 0)

    def out_map(p, b):
        return (jnp.where(p == 1, b, 0), 0, 0)

    in_specs = [
        pl.BlockSpec((1, ngt, nc), in_map),
        pl.BlockSpec((1, 4, na), in_map),
        pl.BlockSpec((1, 2, na), lambda p, b: (0, 0, 0)),
        pl.BlockSpec((1, ngt, 1), in_map),
        pl.BlockSpec((1, ngt, 4), in_map),
        pl.BlockSpec((1, ngt, 1), in_map),
    ]
    out_specs = (
        pl.BlockSpec((1, 1, na), out_map),
        pl.BlockSpec((1, na, 4), out_map),
        pl.BlockSpec((1, na, NC), out_map),
        pl.BlockSpec((1, 1, na), out_map),
        pl.BlockSpec((1, 1, na), out_map),
    )
    scratch = [
        pltpu.VMEM((ngt, na), jnp.float32),
        pltpu.VMEM((ngt, na), jnp.float32),
        pltpu.VMEM((bs * ngt, na), jnp.int8),
        pltpu.VMEM((bs * 8, na), jnp.int32),
        pltpu.SMEM((bs,), jnp.int32),
    ]
    tcls, tb, ts, fm, tg = pl.pallas_call(
        _body,
        grid=grid,
        in_specs=in_specs,
        out_specs=out_specs,
        out_shape=out_shapes,
        scratch_shapes=scratch,
        interpret=_INTERPRET,
    )(ps_s, pb_t, anc_t, gl, gt_bboxes, gt_mask)

    return (tcls.reshape(bs, na), tb, ts,
            fm.reshape(bs, na) > 0, tg.reshape(bs, na))


# speedup vs baseline: 19.5112x; 1.0542x over previous
import math

import jax
import jax.numpy as jnp
from jax.experimental import pallas as pl
from jax.experimental.pallas import tpu as pltpu

TOPK = 13
NC = 80
EPS_IN = 1e-9
EPS_IOU = 1e-7

_INTERPRET = False

_ATAN_COEFFS = (
    9.999999999776e-01, -3.333333207911e-01, 1.999992819090e-01,
    -1.428419858875e-01, 1.109481209099e-01, -8.987009293820e-02,
    7.264201452628e-02, -5.460683193113e-02, 3.458600582713e-02,
    -1.638182583930e-02, 4.961051519167e-03, -7.042539361997e-04,
)


def _atan_pos(x):
    inv = x > 1.0
    z = jnp.where(inv, 1.0 / x, x)
    u = z * z
    p = jnp.full_like(u, _ATAN_COEFFS[-1])
    for c in _ATAN_COEFFS[-2::-1]:
        p = p * u + c
    at = z * p
    return jnp.where(inv, (math.pi / 2) - at, at)


def _body(ps_ref, pb_ref, anc_ref, gl_ref, gtb_ref, gm_ref,
          cls_ref, tb_ref, ts_ref, fm_ref, tg_ref,
          work_ref, isin_ref, ncm_ref, cfl_ref):
    p = pl.program_id(0)
    b = pl.program_id(1)
    bs = pl.num_programs(1)
    ngt = gtb_ref.shape[1]
    na = pb_ref.shape[2]

    gl = gl_ref[0]
    gtb = gtb_ref[0]
    valid = gm_ref[0] > 0.0
    i_iota = jax.lax.broadcasted_iota(jnp.int32, (ngt, na), 1)
    j_iota = jax.lax.broadcasted_iota(jnp.int32, (ngt, na), 0)

    @pl.when(p == 0)
    def _phase0():
        pb = pb_ref[0]
        px1 = pb[0:1, :]
        px2 = pb[1:2, :]
        py1 = pb[2:3, :]
        py2 = pb[3:4, :]
        anc = anc_ref[0]
        ax = anc[0:1, :]
        ay = anc[1:2, :]
        gx1 = gtb[:, 0:1]
        gx2 = gtb[:, 1:2]
        gy1 = gtb[:, 2:3]
        gy2 = gtb[:, 3:4]

        d1 = ax - gx1
        d2 = ay - gx2
        d3 = gy1 - ax
        d4 = gy2 - ay
        dmin = jnp.minimum(jnp.minimum(d1, d2), jnp.minimum(d3, d4))
        mask = jnp.logical_and(dmin > EPS_IN, valid)

        inter = (jnp.clip(jnp.minimum(px2, gx2) - jnp.maximum(px1, gx1), 0.0, None)
                 * jnp.clip(jnp.minimum(py2, gy2) - jnp.maximum(py1, gy1), 0.0, None))
        w1 = px2 - px1
        h1 = py2 - py1 + EPS_IOU
        w2 = gx2 - gx1
        h2 = gy2 - gy1 + EPS_IOU
        union = w1 * h1 + w2 * h2 - inter + EPS_IOU
        iou = inter / union
        cw = jnp.maximum(px2, gx2) - jnp.minimum(px1, gx1)
        ch = jnp.maximum(py2, gy2) - jnp.minimum(py1, gy1)
        c2 = cw * cw + ch * ch + EPS_IOU
        rho2 = ((gx1 + gx2 - px1 - px2) ** 2 + (gy1 + gy2 - py1 - py2) ** 2) / 4.0
        at1 = _atan_pos(w1 / h1)
        at2 = _atan_pos(w2 / h2)
        dat = at2 - at1
        v = (4.0 / math.pi ** 2) * dat * dat
        alpha = v / (v - iou + (1.0 + EPS_IOU))
        ciou = iou - (rho2 / c2 + v * alpha)
        iou_c = jnp.maximum(ciou, 0.0)

        ps = ps_ref[0]
        cls_iota = jax.lax.broadcasted_iota(jnp.int32, ps.shape, 1)
        oh = (cls_iota == gl).astype(ps.dtype)
        s = jnp.sum(ps * oh, axis=1, keepdims=True)

        i2 = iou_c * iou_c
        i6 = i2 * i2 * i2
        align = s * i6 * mask.astype(ps.dtype)

        work_ref[:] = align

        def topk_step(_, carry):
            w = work_ref[:]
            m = jnp.max(w, axis=1, keepdims=True)
            first = jnp.min(jnp.where(w == m, i_iota, na), axis=1,
                            keepdims=True)
            work_ref[:] = jnp.where(i_iota == first, -jnp.inf, w)
            return carry

        jax.lax.fori_loop(0, TOPK, topk_step, 0)
        sel = work_ref[:] != align

        is_in = (sel & valid & (align > 1e-9)).astype(jnp.int32)

        cnt = jnp.sum(is_in, axis=0, keepdims=True)
        conflict = cnt > 1
        amax = jnp.max(align, axis=0, keepdims=True)
        cg = jnp.min(jnp.where(align == amax, j_iota, ngt), axis=0,
                     keepdims=True)
        hit = jnp.max((cg == j_iota).astype(jnp.int32), axis=1,
                      keepdims=True)
        ncm_row = jnp.max(jnp.where(i_iota == j_iota, hit, 0), axis=0,
                          keepdims=True)

        isin_ref[pl.ds(b * ngt, ngt), :] = is_in.astype(jnp.int8)
        ncm_ref[pl.ds(b * 8, 8), :] = jnp.broadcast_to(ncm_row, (8, na))
        cfl_ref[b] = jnp.max(conflict.astype(jnp.int32))

    @pl.when(p == 1)
    def _phase1():
        is_in = isin_ref[pl.ds(b * ngt, ngt), :].astype(jnp.int32)
        ncm_row = ncm_ref[pl.ds(b * 8, 1), :]
        cnt = jnp.sum(is_in, axis=0, keepdims=True)
        conflict = cnt > 1
        nm = jnp.logical_not(conflict).astype(jnp.int32)
        ncm = jnp.where(j_iota == 0, ncm_row, 0)
        resolved = (is_in + ncm) * nm
        anycf = jnp.int32(0)
        for bb in range(bs):
            anycf = anycf + cfl_ref[bb]
        is_f = jnp.where(anycf > 0, resolved, is_in)

        fmask = jnp.sum(is_f, axis=0, keepdims=True) > 0
        mxv = jnp.max(is_f, axis=0, keepdims=True)
        tg = jnp.min(jnp.where(is_f == mxv, j_iota, ngt), axis=0,
                     keepdims=True)

        sel_t = j_iota == tg
        cmp_t = sel_t.astype(jnp.float32)
        glf = gl.astype(jnp.float32)
        tcls = jnp.sum(jnp.where(sel_t, glf, 0.0), axis=0, keepdims=True)

        cls_ref[0] = tcls.astype(jnp.int32)
        fm_ref[0] = fmask.astype(jnp.int32)
        tg_ref[0] = tg

        dn = (((0,), (0,)), ((), ()))
        tb_ref[0] = jax.lax.dot_general(cmp_t, gtb, dn,
                                        preferred_element_type=jnp.float32)
        oh_cls = (jax.lax.broadcasted_iota(jnp.int32, (ngt, NC), 1)
                  == gl).astype(jnp.float32)
        ts_ref[0] = jax.lax.dot_general(cmp_t, oh_cls, dn,
                                        preferred_element_type=jnp.float32)


@jax.jit
def kernel(pd_scores, pd_bboxes, anc_points, gt_labels, gt_bboxes, gt_mask):
    bs, na, nc = pd_scores.shape
    ngt = gt_bboxes.shape[1]

    ps_s = pd_scores[:, :ngt, :]
    pb_t = jnp.transpose(pd_bboxes, (0, 2, 1))
    anc_t = jnp.transpose(anc_points).reshape(1, 2, na)
    gl = gt_labels.astype(jnp.int32)

    out_shapes = (
        jax.ShapeDtypeStruct((bs, 1, na), jnp.int32),
        jax.ShapeDtypeStruct((bs, na, 4), jnp.float32),
        jax.ShapeDtypeStruct((bs, na, NC), jnp.float32),
        jax.ShapeDtypeStruct((bs, 1, na), jnp.int32),
        jax.ShapeDtypeStruct((bs, 1, na), jnp.int32),
    )
    grid = (2, bs)

    def in_map(p, b):
        return (b, 0, 0)

    def out_map(p, b):
        return (jnp.where(p == 1, b, 0), 0, 0)

    in_specs = [
        pl.BlockSpec((1, ngt, nc), in_map),
        pl.BlockSpec((1, 4, na), in_map),
        pl.BlockSpec((1, 2, na), lambda p, b: (0, 0, 0)),
        pl.BlockSpec((1, ngt, 1), in_map),
        pl.BlockSpec((1, ngt, 4), in_map),
        pl.BlockSpec((1, ngt, 1), in_map),
    ]
    out_specs = (
        pl.BlockSpec((1, 1, na), out_map),
        pl.BlockSpec((1, na, 4), out_map),
        pl.BlockSpec((1, na, NC), out_map),
        pl.BlockSpec((1, 1, na), out_map),
        pl.BlockSpec((1, 1, na), out_map),
    )
    scratch = [
        pltpu.VMEM((ngt, na), jnp.float32),
        pltpu.VMEM((bs * ngt, na), jnp.int8),
        pltpu.VMEM((bs * 8, na), jnp.int32),
        pltpu.SMEM((bs,), jnp.int32),
    ]
    tcls, tb, ts, fm, tg = pl.pallas_call(
        _body,
        grid=grid,
        in_specs=in_specs,
        out_specs=out_specs,
        out_shape=out_shapes,
        scratch_shapes=scratch,
        interpret=_INTERPRET,
    )(ps_s, pb_t, anc_t, gl, gt_bboxes, gt_mask)

    return (tcls.reshape(bs, na), tb, ts,
            fm.reshape(bs, na) > 0, tg.reshape(bs, na))


# speedup vs baseline: 21.9786x; 1.1265x over previous
import math

import jax
import jax.numpy as jnp
from jax.experimental import pallas as pl
from jax.experimental.pallas import tpu as pltpu

TOPK = 13
NC = 80
EPS_IN = 1e-9
EPS_IOU = 1e-7

_INTERPRET = False

_ATAN_COEFFS = (
    9.999999999776e-01, -3.333333207911e-01, 1.999992819090e-01,
    -1.428419858875e-01, 1.109481209099e-01, -8.987009293820e-02,
    7.264201452628e-02, -5.460683193113e-02, 3.458600582713e-02,
    -1.638182583930e-02, 4.961051519167e-03, -7.042539361997e-04,
)


def _atan_pos(x):
    inv = x > 1.0
    z = jnp.where(inv, 1.0 / x, x)
    u = z * z
    p = jnp.full_like(u, _ATAN_COEFFS[-1])
    for c in _ATAN_COEFFS[-2::-1]:
        p = p * u + c
    at = z * p
    return jnp.where(inv, (math.pi / 2) - at, at)


def _body(ps_ref, pb_ref, anc_ref, gl_ref, gtb_ref, gm_ref,
          cls_ref, tb_ref, ts_ref, fm_ref, tg_ref,
          work_ref, isin_ref, ncm_ref, cfl_ref):
    p = pl.program_id(0)
    b = pl.program_id(1)
    bs = pl.num_programs(1)
    ngt = gtb_ref.shape[1]
    na = pb_ref.shape[2]

    gl = gl_ref[0]
    gtb = gtb_ref[0]
    valid = gm_ref[0] > 0.0
    i_iota = jax.lax.broadcasted_iota(jnp.int32, (ngt, na), 1)
    j_iota = jax.lax.broadcasted_iota(jnp.int32, (ngt, na), 0)

    @pl.when(p == 0)
    def _phase0():
        pb = pb_ref[0]
        px1 = pb[0:1, :]
        px2 = pb[1:2, :]
        py1 = pb[2:3, :]
        py2 = pb[3:4, :]
        anc = anc_ref[0]
        ax = anc[0:1, :]
        ay = anc[1:2, :]
        gx1 = gtb[:, 0:1]
        gx2 = gtb[:, 1:2]
        gy1 = gtb[:, 2:3]
        gy2 = gtb[:, 3:4]

        d1 = ax - gx1
        d2 = ay - gx2
        d3 = gy1 - ax
        d4 = gy2 - ay
        dmin = jnp.minimum(jnp.minimum(d1, d2), jnp.minimum(d3, d4))
        mask = jnp.logical_and(dmin > EPS_IN, valid)

        inter = (jnp.clip(jnp.minimum(px2, gx2) - jnp.maximum(px1, gx1), 0.0, None)
                 * jnp.clip(jnp.minimum(py2, gy2) - jnp.maximum(py1, gy1), 0.0, None))
        w1 = px2 - px1
        h1 = py2 - py1 + EPS_IOU
        w2 = gx2 - gx1
        h2 = gy2 - gy1 + EPS_IOU
        union = w1 * h1 + w2 * h2 - inter + EPS_IOU
        iou = inter / union
        cw = jnp.maximum(px2, gx2) - jnp.minimum(px1, gx1)
        ch = jnp.maximum(py2, gy2) - jnp.minimum(py1, gy1)
        c2 = cw * cw + ch * ch + EPS_IOU
        rho2 = ((gx1 + gx2 - px1 - px2) ** 2 + (gy1 + gy2 - py1 - py2) ** 2) / 4.0
        at1 = _atan_pos(w1 / h1)
        at2 = _atan_pos(w2 / h2)
        dat = at2 - at1
        v = (4.0 / math.pi ** 2) * dat * dat
        alpha = v / (v - iou + (1.0 + EPS_IOU))
        ciou = iou - (rho2 / c2 + v * alpha)
        iou_c = jnp.maximum(ciou, 0.0)

        ps = ps_ref[0]
        cls_iota = jax.lax.broadcasted_iota(jnp.int32, ps.shape, 1)
        oh = (cls_iota == gl).astype(ps.dtype)
        s = jnp.sum(ps * oh, axis=1, keepdims=True)

        i2 = iou_c * iou_c
        i6 = i2 * i2 * i2
        align = s * i6 * mask.astype(ps.dtype)

        work_ref[:] = align

        w = align
        for _ in range(TOPK):
            m = jnp.max(w, axis=1, keepdims=True)
            first = jnp.min(jnp.where(w == m, i_iota, na), axis=1,
                            keepdims=True)
            w = jnp.where(i_iota == first, -jnp.inf, w)
        work_ref[:] = w
        sel = work_ref[:] != align

        is_in = (sel & valid & (align > 1e-9)).astype(jnp.int32)

        cnt = jnp.sum(is_in, axis=0, keepdims=True)
        conflict = cnt > 1
        amax = jnp.max(align, axis=0, keepdims=True)
        cg = jnp.min(jnp.where(align == amax, j_iota, ngt), axis=0,
                     keepdims=True)
        hit = jnp.max((cg == j_iota).astype(jnp.int32), axis=1,
                      keepdims=True)
        ncm_row = jnp.max(jnp.where(i_iota == j_iota, hit, 0), axis=0,
                          keepdims=True)

        isin_ref[pl.ds(b * ngt, ngt), :] = is_in.astype(jnp.int8)
        ncm_ref[pl.ds(b * 8, 8), :] = jnp.broadcast_to(ncm_row, (8, na))
        cfl_ref[b] = jnp.max(conflict.astype(jnp.int32))

    @pl.when(p == 1)
    def _phase1():
        is_in = isin_ref[pl.ds(b * ngt, ngt), :].astype(jnp.int32)
        ncm_row = ncm_ref[pl.ds(b * 8, 1), :]
        cnt = jnp.sum(is_in, axis=0, keepdims=True)
        conflict = cnt > 1
        nm = jnp.logical_not(conflict).astype(jnp.int32)
        ncm = jnp.where(j_iota == 0, ncm_row, 0)
        resolved = (is_in + ncm) * nm
        anycf = jnp.int32(0)
        for bb in range(bs):
            anycf = anycf + cfl_ref[bb]
        is_f = jnp.where(anycf > 0, resolved, is_in)

        fmask = jnp.sum(is_f, axis=0, keepdims=True) > 0
        mxv = jnp.max(is_f, axis=0, keepdims=True)
        tg = jnp.min(jnp.where(is_f == mxv, j_iota, ngt), axis=0,
                     keepdims=True)

        sel_t = j_iota == tg
        cmp_t = sel_t.astype(jnp.float32)
        glf = gl.astype(jnp.float32)
        tcls = jnp.sum(jnp.where(sel_t, glf, 0.0), axis=0, keepdims=True)

        cls_ref[0] = tcls.astype(jnp.int32)
        fm_ref[0] = fmask.astype(jnp.int32)
        tg_ref[0] = tg

        dn = (((0,), (0,)), ((), ()))
        tb_ref[0] = jax.lax.dot_general(cmp_t, gtb, dn,
                                        preferred_element_type=jnp.float32)
        oh_cls = (jax.lax.broadcasted_iota(jnp.int32, (ngt, NC), 1)
                  == gl).astype(jnp.float32)
        ts_ref[0] = jax.lax.dot_general(cmp_t, oh_cls, dn,
                                        preferred_element_type=jnp.float32)


@jax.jit
def kernel(pd_scores, pd_bboxes, anc_points, gt_labels, gt_bboxes, gt_mask):
    bs, na, nc = pd_scores.shape
    ngt = gt_bboxes.shape[1]

    ps_s = pd_scores[:, :ngt, :]
    pb_t = jnp.transpose(pd_bboxes, (0, 2, 1))
    anc_t = jnp.transpose(anc_points).reshape(1, 2, na)
    gl = gt_labels.astype(jnp.int32)

    out_shapes = (
        jax.ShapeDtypeStruct((bs, 1, na), jnp.int32),
        jax.ShapeDtypeStruct((bs, na, 4), jnp.float32),
        jax.ShapeDtypeStruct((bs, na, NC), jnp.float32),
        jax.ShapeDtypeStruct((bs, 1, na), jnp.int32),
        jax.ShapeDtypeStruct((bs, 1, na), jnp.int32),
    )
    grid = (2, bs)

    def in_map(p, b):
        return (b, 0, 0)

    def out_map(p, b):
        return (jnp.where(p == 1, b, 0), 0, 0)

    in_specs = [
        pl.BlockSpec((1, ngt, nc), in_map),
        pl.BlockSpec((1, 4, na), in_map),
        pl.BlockSpec((1, 2, na), lambda p, b: (0, 0, 0)),
        pl.BlockSpec((1, ngt, 1), in_map),
        pl.BlockSpec((1, ngt, 4), in_map),
        pl.BlockSpec((1, ngt, 1), in_map),
    ]
    out_specs = (
        pl.BlockSpec((1, 1, na), out_map),
        pl.BlockSpec((1, na, 4), out_map),
        pl.BlockSpec((1, na, NC), out_map),
        pl.BlockSpec((1, 1, na), out_map),
        pl.BlockSpec((1, 1, na), out_map),
    )
    scratch = [
        pltpu.VMEM((ngt, na), jnp.float32),
        pltpu.VMEM((bs * ngt, na), jnp.int8),
        pltpu.VMEM((bs * 8, na), jnp.int32),
        pltpu.SMEM((bs,), jnp.int32),
    ]
    tcls, tb, ts, fm, tg = pl.pallas_call(
        _body,
        grid=grid,
        in_specs=in_specs,
        out_specs=out_specs,
        out_shape=out_shapes,
        scratch_shapes=scratch,
        interpret=_INTERPRET,
    )(ps_s, pb_t, anc_t, gl, gt_bboxes, gt_mask)

    return (tcls.reshape(bs, na), tb, ts,
            fm.reshape(bs, na) > 0, tg.reshape(bs, na))
